# baseline (speedup 1.0000x reference)
import functools

import jax
import jax.numpy as jnp
from jax import lax
from jax.experimental import pallas as pl
from jax.experimental.pallas import tpu as pltpu
from jax.experimental.pallas import tpu_sc as plsc

N = 10000
NP = 10240
E = 160000
XD = 256
ED = 16
H = 256
GW = 512
GWP = 256

NC, NS = 2, 16
NW = NC * NS
KG = 104
GCH = 50
EPW = KG * GCH
EPAD = NW * EPW

CPW = H // NW
KS = 640
SCH = E // KS


def _tables_body(x_ref, wa_ref, wb_ref, ba_ref, a_ref, b_ref):
    xb = x_ref[...]
    a_ref[...] = (jnp.dot(xb, wa_ref[...], preferred_element_type=jnp.float32)
                  + ba_ref[...]).astype(jnp.bfloat16)
    b_ref[...] = jnp.dot(xb, wb_ref[...],
                         preferred_element_type=jnp.float32).astype(jnp.bfloat16)


def _tables(x2, WA, WB, bA):
    BN = 2048
    return pl.pallas_call(
        _tables_body,
        grid=(NP // BN,),
        in_specs=[
            pl.BlockSpec((BN, XD), lambda i: (i, 0)),
            pl.BlockSpec((XD, GW), lambda i: (0, 0)),
            pl.BlockSpec((XD, GW), lambda i: (0, 0)),
            pl.BlockSpec((1, GW), lambda i: (0, 0)),
        ],
        out_specs=[
            pl.BlockSpec((BN, GW), lambda i: (i, 0)),
            pl.BlockSpec((BN, GW), lambda i: (i, 0)),
        ],
        out_shape=[jax.ShapeDtypeStruct((NP, GW), jnp.bfloat16)] * 2,
    )(x2, WA, WB, bA)


def _sc_gather(A, B, recv_pad, send_pad):
    mesh = plsc.VectorSubcoreMesh(core_axis_name="c", subcore_axis_name="s")

    @functools.partial(
        pl.kernel,
        mesh=mesh,
        out_type=(jax.ShapeDtypeStruct((EPAD, GWP), jnp.float32),
                  jax.ShapeDtypeStruct((EPAD, GWP), jnp.float32)),
        scratch_types=[
            pltpu.VMEM((2, KG), jnp.int32),
            pltpu.VMEM((2, KG), jnp.int32),
            pltpu.VMEM((2, KG, GWP), jnp.float32),
            pltpu.VMEM((2, KG, GWP), jnp.float32),
            [pltpu.SemaphoreType.DMA] * 2,
            [pltpu.SemaphoreType.DMA] * 2,
            [pltpu.SemaphoreType.DMA] * 2,
            [pltpu.SemaphoreType.DMA] * 2,
        ],
    )
    def k(a_hbm, b_hbm, recv_hbm, send_hbm, outa_hbm, outb_hbm,
          idxr, idxs, bufa, bufb, semi, sema, semb, semo):
        wid = lax.axis_index("s") * NC + lax.axis_index("c")
        base = wid * EPW

        def issue_idx(c, b):
            pltpu.async_copy(recv_hbm.at[pl.ds(base + c * KG, KG)],
                             idxr.at[b], semi[b])
            pltpu.async_copy(send_hbm.at[pl.ds(base + c * KG, KG)],
                             idxs.at[b], semi[b])

        issue_idx(0, 0)
        issue_idx(1, 1)

        def pair(p, carry):
            for b in range(2):
                c = p * 2 + b
                pltpu.make_async_copy(recv_hbm.at[pl.ds(0, KG)],
                                      idxr.at[b], semi[b]).wait()
                pltpu.make_async_copy(send_hbm.at[pl.ds(0, KG)],
                                      idxs.at[b], semi[b]).wait()

                @pl.when(c >= 2)
                def _():
                    pltpu.make_async_copy(
                        bufa.at[b], outa_hbm.at[pl.ds(base, KG)],
                        semo[b]).wait()
                    pltpu.make_async_copy(
                        bufb.at[b], outb_hbm.at[pl.ds(base, KG)],
                        semo[b]).wait()

                ca = pltpu.async_copy(a_hbm.at[idxr.at[b]], bufa.at[b],
                                      sema[b])
                cb = pltpu.async_copy(b_hbm.at[idxs.at[b]], bufb.at[b],
                                      semb[b])

                @pl.when(c + 2 < GCH)
                def _():
                    issue_idx(c + 2, b)

                ca.wait()
                cb.wait()
                pltpu.async_copy(bufa.at[b],
                                 outa_hbm.at[pl.ds(base + c * KG, KG)],
                                 semo[b])
                pltpu.async_copy(bufb.at[b],
                                 outb_hbm.at[pl.ds(base + c * KG, KG)],
                                 semo[b])
            return carry

        lax.fori_loop(0, GCH // 2, pair, 0)
        for b in range(2):
            pltpu.make_async_copy(bufa.at[b], outa_hbm.at[pl.ds(base, KG)],
                                  semo[b]).wait()
            pltpu.make_async_copy(bufb.at[b], outb_hbm.at[pl.ds(base, KG)],
                                  semo[b]).wait()

    return k(A, B, recv_pad, send_pad)


def _unpack_pair(ref):
    w = lax.bitcast_convert_type(ref[...], jnp.int32)
    lo = lax.bitcast_convert_type(w << 16, jnp.float32)
    hi = lax.bitcast_convert_type(w & jnp.int32(-65536), jnp.float32)
    return lo, hi


def _edge_body(ga_ref, gb_ref, e_ref, we_ref, pw2_ref, pb2_ref, f_ref,
               fw2_ref, fb2_ref, eout_ref, mt_ref):
    lo_a, hi_a = _unpack_pair(ga_ref)
    lo_b, hi_b = _unpack_pair(gb_ref)
    lo = lo_a + lo_b
    hi = hi_a + hi_b
    HH = H // 2
    g1 = jnp.concatenate([lo[:, :HH], hi[:, :HH]], axis=1)
    g2 = jnp.concatenate([lo[:, HH:], hi[:, HH:]], axis=1)
    h1 = jnp.maximum(
        g1 + jnp.dot(e_ref[...], we_ref[...], preferred_element_type=jnp.float32),
        0.0)
    eo = jnp.dot(h1, pw2_ref[...], preferred_element_type=jnp.float32) + pb2_ref[...]
    eout_ref[...] = eo
    h2 = jnp.maximum(
        g2 + jnp.dot(eo, f_ref[...], preferred_element_type=jnp.float32),
        0.0)
    mt = lax.dot_general(fw2_ref[...], h2, (((0,), (1,)), ((), ())),
                         preferred_element_type=jnp.float32)
    mt_ref[...] = mt + fb2_ref[...]


def _edge_mlp(Ga, Gb, e2, We, pW2, pb2, F, fW2, fb2col):
    TE = 1280
    return pl.pallas_call(
        _edge_body,
        grid=(E // TE,),
        in_specs=[
            pl.BlockSpec((TE, GWP), lambda i: (i, 0)),
            pl.BlockSpec((TE, GWP), lambda i: (i, 0)),
            pl.BlockSpec((TE, ED), lambda i: (i, 0)),
            pl.BlockSpec((ED, H), lambda i: (0, 0)),
            pl.BlockSpec((H, ED), lambda i: (0, 0)),
            pl.BlockSpec((1, ED), lambda i: (0, 0)),
            pl.BlockSpec((ED, H), lambda i: (0, 0)),
            pl.BlockSpec((H, H), lambda i: (0, 0)),
            pl.BlockSpec((H, 1), lambda i: (0, 0)),
        ],
        out_specs=[
            pl.BlockSpec((TE, ED), lambda i: (i, 0)),
            pl.BlockSpec((H, TE), lambda i: (0, i)),
        ],
        out_shape=[
            jax.ShapeDtypeStruct((E, ED), jnp.float32),
            jax.ShapeDtypeStruct((H, E), jnp.float32),
        ],
    )(Ga, Gb, e2, We, pW2, pb2, F, fW2, fb2col)


def _sc_scatter(mT, recv_flat):
    mesh = plsc.VectorSubcoreMesh(core_axis_name="c", subcore_axis_name="s")

    @functools.partial(
        pl.kernel,
        mesh=mesh,
        out_type=jax.ShapeDtypeStruct((H * NP,), jnp.float32),
        compiler_params=pltpu.CompilerParams(needs_layout_passes=False),
        scratch_types=[
            pltpu.VMEM((KS,), jnp.int32),
            pltpu.VMEM((CPW, KS), jnp.float32),
            pltpu.VMEM((CPW * NP,), jnp.float32),
            pltpu.SemaphoreType.DMA,
        ],
    )
    def k(mt_hbm, r_hbm, aggt_hbm, idxb, mbuf, acc, sem):
        wid = lax.axis_index("s") * NC + lax.axis_index("c")
        zeros = jnp.zeros((16,), jnp.float32)

        def zchunk(j, c2):
            for v in range(16):
                acc[pl.ds((j * 16 + v) * 16, 16)] = zeros
            return c2

        lax.fori_loop(0, CPW * NP // 256, zchunk, 0)

        def chunk(ci, carry):
            off = ci * KS
            pltpu.sync_copy(r_hbm.at[pl.ds(off, KS)], idxb)
            cm = pltpu.async_copy(
                mt_hbm.at[pl.ds(wid * CPW, CPW), pl.ds(off, KS)], mbuf, sem)
            cm.wait()

            def group(g, c2):
                ids = idxb[pl.ds(g * 16, 16)]
                for c in range(CPW):
                    data = mbuf[c, pl.ds(g * 16, 16)]
                    plsc.addupdate_scatter(acc, [ids + (c * NP)], data)
                return c2

            lax.fori_loop(0, KS // 16, group, 0)
            return carry

        lax.fori_loop(0, SCH, chunk, 0)
        pltpu.sync_copy(acc, aggt_hbm.at[pl.ds(wid * CPW * NP, CPW * NP)])

    return k(mT, recv_flat)


def _node_body(x_ref, aggt_ref, na_ref, nb_ref, b1_ref, w2_ref, b2_ref,
               out_ref):
    agg_contrib = lax.dot_general(aggt_ref[...], nb_ref[...],
                                  (((0,), (0,)), ((), ())),
                                  preferred_element_type=jnp.float32)
    h = jnp.maximum(
        jnp.dot(x_ref[...], na_ref[...], preferred_element_type=jnp.float32)
        + agg_contrib + b1_ref[...], 0.0)
    out_ref[...] = jnp.dot(h, w2_ref[...], preferred_element_type=jnp.float32) + b2_ref[...]


def _node_mlp(x2, aggT, nuA, nuB, nb1, nW2, nb2):
    BN = 2048
    return pl.pallas_call(
        _node_body,
        grid=(NP // BN,),
        in_specs=[
            pl.BlockSpec((BN, XD), lambda i: (i, 0)),
            pl.BlockSpec((H, BN), lambda i: (0, i)),
            pl.BlockSpec((XD, H), lambda i: (0, 0)),
            pl.BlockSpec((H, H), lambda i: (0, 0)),
            pl.BlockSpec((1, H), lambda i: (0, 0)),
            pl.BlockSpec((H, XD), lambda i: (0, 0)),
            pl.BlockSpec((1, XD), lambda i: (0, 0)),
        ],
        out_specs=pl.BlockSpec((BN, XD), lambda i: (i, 0)),
        out_shape=jax.ShapeDtypeStruct((NP, XD), jnp.float32),
    )(x2, aggT, nuA, nuB, nb1, nW2, nb2)


def kernel(x, e, edge_index, psi_W1, psi_b1, psi_W2, psi_b2,
           phi_W1, phi_b1, phi_W2, phi_b2, nu_W1, nu_b1, nu_W2, nu_b2):
    x2 = jnp.pad(x.reshape(N, XD), ((0, NP - N), (0, 0)))
    e2 = e.reshape(E, ED)
    ei = edge_index.astype(jnp.int32)
    send_flat = ei[0]
    recv_flat = ei[1]

    perm_blk = jnp.stack(
        [jnp.arange(H // 2), jnp.arange(H // 2) + H // 2], axis=1).reshape(H)
    perm = jnp.concatenate([perm_blk, perm_blk + H])
    We = psi_W1[:ED]
    WA = jnp.concatenate([psi_W1[ED:ED + XD], phi_W1[:XD]], axis=1)[:, perm]
    WB = jnp.concatenate([psi_W1[ED + XD:], phi_W1[XD:2 * XD]], axis=1)[:, perm]
    bA = jnp.concatenate([psi_b1, phi_b1])[perm].reshape(1, GW)
    F = phi_W1[2 * XD:]

    pad = jnp.full((EPAD - E,), N, jnp.int32)
    recv_pad = jnp.concatenate([recv_flat, pad])
    send_pad = jnp.concatenate([send_flat, pad])

    A, B = _tables(x2, WA, WB, bA)
    Ap = lax.bitcast_convert_type(A.reshape(NP, GWP, 2), jnp.float32)
    Bp = lax.bitcast_convert_type(B.reshape(NP, GWP, 2), jnp.float32)
    Ga, Gb = _sc_gather(Ap, Bp, recv_pad, send_pad)
    e_out, mT = _edge_mlp(Ga, Gb, e2, We, psi_W2, psi_b2.reshape(1, ED),
                          F, phi_W2, phi_b2.reshape(H, 1))
    aggT = _sc_scatter(mT, recv_flat).reshape(H, NP)
    x_out = _node_mlp(x2, aggT, nu_W1[:XD], nu_W1[XD:], nu_b1.reshape(1, H),
                      nu_W2, nu_b2.reshape(1, XD))
    return (x_out[:N].reshape(1, N, XD), e_out.reshape(1, E, ED))


# speedup vs baseline: 2.0194x; 2.0194x over previous
import functools

import jax
import jax.numpy as jnp
from jax import lax
from jax.experimental import pallas as pl
from jax.experimental.pallas import tpu as pltpu
from jax.experimental.pallas import tpu_sc as plsc

N = 10000
NP = 10240
E = 160000
XD = 256
ED = 16
H = 256
GW = 512
GWP = 256

NC, NS = 2, 16
NW = NC * NS
KG = 104
GCH = 48
EPW = KG * GCH
ETAIL = NW * EPW

CPW = H // NW
KS = 640
SCH = E // KS


def _mask16():
    return jnp.int32(-65536)


def _pack_bf16_pairs(v):
    HH = H // 2
    lo = jnp.concatenate([v[:, 0:HH], v[:, 2 * HH:3 * HH]], axis=1)
    hi = jnp.concatenate([v[:, HH:2 * HH], v[:, 3 * HH:4 * HH]], axis=1)
    wl = lax.bitcast_convert_type(lo, jnp.int32)
    wh = lax.bitcast_convert_type(hi, jnp.int32)
    rl = wl + jnp.int32(0x7FFF) + ((wl >> 16) & 1)
    rh = wh + jnp.int32(0x7FFF) + ((wh >> 16) & 1)
    word = ((rl >> 16) & jnp.int32(0xFFFF)) | (rh & _mask16())
    return lax.bitcast_convert_type(word, jnp.float32)


def _tables_body(x_ref, wa_ref, wb_ref, ba_ref, a_ref, b_ref):
    xb = x_ref[...]
    a = jnp.dot(xb, wa_ref[...], preferred_element_type=jnp.float32) + ba_ref[...]
    bt = jnp.dot(xb, wb_ref[...], preferred_element_type=jnp.float32)
    a_ref[...] = _pack_bf16_pairs(a)
    b_ref[...] = _pack_bf16_pairs(bt)


def _tables(x2, WA, WB, bA):
    BN = 2000
    return pl.pallas_call(
        _tables_body,
        grid=(N // BN,),
        in_specs=[
            pl.BlockSpec((BN, XD), lambda i: (i, 0)),
            pl.BlockSpec((XD, GW), lambda i: (0, 0)),
            pl.BlockSpec((XD, GW), lambda i: (0, 0)),
            pl.BlockSpec((1, GW), lambda i: (0, 0)),
        ],
        out_specs=[
            pl.BlockSpec((BN, GWP), lambda i: (i, 0)),
            pl.BlockSpec((BN, GWP), lambda i: (i, 0)),
        ],
        out_shape=[jax.ShapeDtypeStruct((N, GWP), jnp.float32)] * 2,
    )(x2, WA, WB, bA)


def _sc_gather(A, B, recv, send):
    mesh = plsc.VectorSubcoreMesh(core_axis_name="c", subcore_axis_name="s")

    @functools.partial(
        pl.kernel,
        mesh=mesh,
        out_type=(jax.ShapeDtypeStruct((E, GWP), jnp.float32),
                  jax.ShapeDtypeStruct((E, GWP), jnp.float32)),
        scratch_types=[
            pltpu.VMEM((2, KG), jnp.int32),
            pltpu.VMEM((2, KG), jnp.int32),
            pltpu.VMEM((2, KG, GWP), jnp.float32),
            pltpu.VMEM((2, KG, GWP), jnp.float32),
            [pltpu.SemaphoreType.DMA] * 2,
            [pltpu.SemaphoreType.DMA] * 2,
            [pltpu.SemaphoreType.DMA] * 2,
            [pltpu.SemaphoreType.DMA] * 2,
        ],
    )
    def k(a_hbm, b_hbm, recv_hbm, send_hbm, outa_hbm, outb_hbm,
          idxr, idxs, bufa, bufb, semi, sema, semb, semo):
        wid = lax.axis_index("s") * NC + lax.axis_index("c")
        base = wid * EPW

        def issue_idx(c, b):
            pltpu.async_copy(recv_hbm.at[pl.ds(base + c * KG, KG)],
                             idxr.at[b], semi[b])
            pltpu.async_copy(send_hbm.at[pl.ds(base + c * KG, KG)],
                             idxs.at[b], semi[b])

        issue_idx(0, 0)
        issue_idx(1, 1)

        def pair(p, carry):
            for b in range(2):
                c = p * 2 + b
                pltpu.make_async_copy(recv_hbm.at[pl.ds(0, KG)],
                                      idxr.at[b], semi[b]).wait()
                pltpu.make_async_copy(send_hbm.at[pl.ds(0, KG)],
                                      idxs.at[b], semi[b]).wait()

                @pl.when(c >= 2)
                def _():
                    pltpu.make_async_copy(
                        bufa.at[b], outa_hbm.at[pl.ds(base, KG)],
                        semo[b]).wait()
                    pltpu.make_async_copy(
                        bufb.at[b], outb_hbm.at[pl.ds(base, KG)],
                        semo[b]).wait()

                ca = pltpu.async_copy(a_hbm.at[idxr.at[b]], bufa.at[b],
                                      sema[b])
                cb = pltpu.async_copy(b_hbm.at[idxs.at[b]], bufb.at[b],
                                      semb[b])

                @pl.when(c + 2 < GCH)
                def _():
                    issue_idx(c + 2, b)

                ca.wait()
                cb.wait()
                pltpu.async_copy(bufa.at[b],
                                 outa_hbm.at[pl.ds(base + c * KG, KG)],
                                 semo[b])
                pltpu.async_copy(bufb.at[b],
                                 outb_hbm.at[pl.ds(base + c * KG, KG)],
                                 semo[b])
            return carry

        lax.fori_loop(0, GCH // 2, pair, 0)
        for b in range(2):
            pltpu.make_async_copy(bufa.at[b], outa_hbm.at[pl.ds(base, KG)],
                                  semo[b]).wait()
            pltpu.make_async_copy(bufb.at[b], outb_hbm.at[pl.ds(base, KG)],
                                  semo[b]).wait()

        @pl.when(wid <= 1)
        def _tail_full():
            toff = ETAIL + wid * KG
            pltpu.sync_copy(recv_hbm.at[pl.ds(toff, KG)], idxr.at[0])
            pltpu.sync_copy(send_hbm.at[pl.ds(toff, KG)], idxs.at[0])
            pltpu.async_copy(a_hbm.at[idxr.at[0]], bufa.at[0], sema[0]).wait()
            pltpu.async_copy(b_hbm.at[idxs.at[0]], bufb.at[0], semb[0]).wait()
            pltpu.sync_copy(bufa.at[0], outa_hbm.at[pl.ds(toff, KG)])
            pltpu.sync_copy(bufb.at[0], outb_hbm.at[pl.ds(toff, KG)])

        @pl.when(wid == 2)
        def _tail_rem():
            KT = 48
            toff = ETAIL + 2 * KG
            pltpu.sync_copy(recv_hbm.at[pl.ds(toff, KT)],
                            idxr.at[0, pl.ds(0, KT)])
            pltpu.sync_copy(send_hbm.at[pl.ds(toff, KT)],
                            idxs.at[0, pl.ds(0, KT)])
            pltpu.async_copy(a_hbm.at[idxr.at[0, pl.ds(0, KT)]],
                             bufa.at[0, pl.ds(0, KT)], sema[0]).wait()
            pltpu.async_copy(b_hbm.at[idxs.at[0, pl.ds(0, KT)]],
                             bufb.at[0, pl.ds(0, KT)], semb[0]).wait()
            pltpu.sync_copy(bufa.at[0, pl.ds(0, KT)],
                            outa_hbm.at[pl.ds(toff, KT)])
            pltpu.sync_copy(bufb.at[0, pl.ds(0, KT)],
                            outb_hbm.at[pl.ds(toff, KT)])

    return k(A, B, recv, send)


def _unpack_pair(ref):
    w = lax.bitcast_convert_type(ref[...], jnp.int32)
    lo = lax.bitcast_convert_type(w << 16, jnp.float32)
    hi = lax.bitcast_convert_type(w & _mask16(), jnp.float32)
    return lo, hi


def _edge_body(ga_ref, gb_ref, e_ref, we_ref, pw2_ref, pb2_ref, f_ref,
               fw2_ref, fb2_ref, eout_ref, mt_ref):
    lo_a, hi_a = _unpack_pair(ga_ref)
    lo_b, hi_b = _unpack_pair(gb_ref)
    lo = lo_a + lo_b
    hi = hi_a + hi_b
    HH = H // 2
    g1 = jnp.concatenate([lo[:, :HH], hi[:, :HH]], axis=1)
    g2 = jnp.concatenate([lo[:, HH:], hi[:, HH:]], axis=1)
    h1 = jnp.maximum(
        g1 + jnp.dot(e_ref[...], we_ref[...], preferred_element_type=jnp.float32),
        0.0)
    eo = jnp.dot(h1, pw2_ref[...], preferred_element_type=jnp.float32) + pb2_ref[...]
    eout_ref[...] = eo
    h2 = jnp.maximum(
        g2 + jnp.dot(eo, f_ref[...], preferred_element_type=jnp.float32),
        0.0)
    mt = lax.dot_general(fw2_ref[...], h2, (((0,), (1,)), ((), ())),
                         preferred_element_type=jnp.float32)
    mt_ref[...] = mt + fb2_ref[...]


def _edge_mlp(Ga, Gb, e2, We, pW2, pb2, F, fW2, fb2col):
    TE = 1280
    return pl.pallas_call(
        _edge_body,
        grid=(E // TE,),
        in_specs=[
            pl.BlockSpec((TE, GWP), lambda i: (i, 0)),
            pl.BlockSpec((TE, GWP), lambda i: (i, 0)),
            pl.BlockSpec((TE, ED), lambda i: (i, 0)),
            pl.BlockSpec((ED, H), lambda i: (0, 0)),
            pl.BlockSpec((H, ED), lambda i: (0, 0)),
            pl.BlockSpec((1, ED), lambda i: (0, 0)),
            pl.BlockSpec((ED, H), lambda i: (0, 0)),
            pl.BlockSpec((H, H), lambda i: (0, 0)),
            pl.BlockSpec((H, 1), lambda i: (0, 0)),
        ],
        out_specs=[
            pl.BlockSpec((TE, ED), lambda i: (i, 0)),
            pl.BlockSpec((H, TE), lambda i: (0, i)),
        ],
        out_shape=[
            jax.ShapeDtypeStruct((E, ED), jnp.float32),
            jax.ShapeDtypeStruct((H, E), jnp.float32),
        ],
    )(Ga, Gb, e2, We, pW2, pb2, F, fW2, fb2col)


def _sc_scatter(mT, recv):
    mesh = plsc.VectorSubcoreMesh(core_axis_name="c", subcore_axis_name="s")

    @functools.partial(
        pl.kernel,
        mesh=mesh,
        out_type=jax.ShapeDtypeStruct((H * NP,), jnp.float32),
        compiler_params=pltpu.CompilerParams(needs_layout_passes=False),
        scratch_types=[
            pltpu.VMEM((2, KS), jnp.int32),
            pltpu.VMEM((2, CPW, KS), jnp.float32),
            pltpu.VMEM((CPW * NP,), jnp.float32),
            [pltpu.SemaphoreType.DMA] * 2,
            [pltpu.SemaphoreType.DMA] * 2,
        ],
    )
    def k(mt_hbm, r_hbm, aggt_hbm, idxb, mbuf, acc, semi, semm):
        wid = lax.axis_index("s") * NC + lax.axis_index("c")
        zeros = jnp.zeros((16,), jnp.float32)

        def issue(c, b):
            pltpu.async_copy(r_hbm.at[pl.ds(c * KS, KS)], idxb.at[b],
                             semi[b])
            pltpu.async_copy(
                mt_hbm.at[pl.ds(wid * CPW, CPW), pl.ds(c * KS, KS)],
                mbuf.at[b], semm[b])

        issue(0, 0)
        issue(1, 1)

        def zchunk(j, c2):
            for v in range(16):
                acc[pl.ds((j * 16 + v) * 16, 16)] = zeros
            return c2

        lax.fori_loop(0, CPW * NP // 256, zchunk, 0)

        def pair(p, carry):
            for b in range(2):
                c = p * 2 + b
                pltpu.make_async_copy(r_hbm.at[pl.ds(0, KS)], idxb.at[b],
                                      semi[b]).wait()
                pltpu.make_async_copy(
                    mt_hbm.at[pl.ds(0, CPW), pl.ds(0, KS)], mbuf.at[b],
                    semm[b]).wait()

                def group(g, c2):
                    ids = idxb[b, pl.ds(g * 16, 16)]
                    for ci in range(CPW):
                        data = mbuf[b, ci, pl.ds(g * 16, 16)]
                        plsc.addupdate_scatter(acc, [ids + (ci * NP)], data)
                    return c2

                lax.fori_loop(0, KS // 16, group, 0)

                @pl.when(c + 2 < SCH)
                def _():
                    issue(c + 2, b)
            return carry

        lax.fori_loop(0, SCH // 2, pair, 0)
        pltpu.sync_copy(acc, aggt_hbm.at[pl.ds(wid * CPW * NP, CPW * NP)])

    return k(mT, recv)


def _node_body(x_ref, aggt_ref, na_ref, nb_ref, b1_ref, w2_ref, b2_ref,
               out_ref):
    agg_contrib = lax.dot_general(aggt_ref[...], nb_ref[...],
                                  (((0,), (0,)), ((), ())),
                                  preferred_element_type=jnp.float32)
    h = jnp.maximum(
        jnp.dot(x_ref[...], na_ref[...], preferred_element_type=jnp.float32)
        + agg_contrib + b1_ref[...], 0.0)
    out_ref[...] = jnp.dot(h, w2_ref[...], preferred_element_type=jnp.float32) + b2_ref[...]


def _node_mlp(x2, aggT, nuA, nuB, nb1, nW2, nb2):
    BN = 2048
    return pl.pallas_call(
        _node_body,
        grid=(NP // BN,),
        in_specs=[
            pl.BlockSpec((BN, XD), lambda i: (i, 0)),
            pl.BlockSpec((H, BN), lambda i: (0, i)),
            pl.BlockSpec((XD, H), lambda i: (0, 0)),
            pl.BlockSpec((H, H), lambda i: (0, 0)),
            pl.BlockSpec((1, H), lambda i: (0, 0)),
            pl.BlockSpec((H, XD), lambda i: (0, 0)),
            pl.BlockSpec((1, XD), lambda i: (0, 0)),
        ],
        out_specs=pl.BlockSpec((BN, XD), lambda i: (i, 0)),
        out_shape=jax.ShapeDtypeStruct((N, XD), jnp.float32),
    )(x2, aggT, nuA, nuB, nb1, nW2, nb2)


def kernel(x, e, edge_index, psi_W1, psi_b1, psi_W2, psi_b2,
           phi_W1, phi_b1, phi_W2, phi_b2, nu_W1, nu_b1, nu_W2, nu_b2):
    x2 = x.reshape(N, XD)
    e2 = e.reshape(E, ED)
    ei = edge_index.astype(jnp.int32)
    send = ei[0]
    recv = ei[1]

    We = psi_W1[:ED]
    WA = jnp.concatenate([psi_W1[ED:ED + XD], phi_W1[:XD]], axis=1)
    WB = jnp.concatenate([psi_W1[ED + XD:], phi_W1[XD:2 * XD]], axis=1)
    bA = jnp.concatenate([psi_b1, phi_b1]).reshape(1, GW)
    F = phi_W1[2 * XD:]

    A, B = _tables(x2, WA, WB, bA)
    Ga, Gb = _sc_gather(A, B, recv, send)
    e_out, mT = _edge_mlp(Ga, Gb, e2, We, psi_W2, psi_b2.reshape(1, ED),
                          F, phi_W2, phi_b2.reshape(H, 1))
    aggT = _sc_scatter(mT, recv).reshape(H, NP)
    x_out = _node_mlp(x2, aggT, nu_W1[:XD], nu_W1[XD:], nu_b1.reshape(1, H),
                      nu_W2, nu_b2.reshape(1, XD))
    return (x_out.reshape(1, N, XD), e_out.reshape(1, E, ED))


# speedup vs baseline: 2.4763x; 1.2262x over previous
import functools

import jax
import jax.numpy as jnp
from jax import lax
from jax.experimental import pallas as pl
from jax.experimental.pallas import tpu as pltpu
from jax.experimental.pallas import tpu_sc as plsc

N = 10000
NP = 10240
E = 160000
XD = 256
ED = 16
H = 256
GW = 512
GWP = 256

NC, NS = 2, 16
NW = NC * NS
KG = 104
GCH = 48
EPW = KG * GCH
ETAIL = NW * EPW

CPW = H // NW
KS = 640
SCH = E // KS


def _mask16():
    return jnp.int32(-65536)


def _pack_bf16_pairs(v):
    HH = H // 2
    lo = jnp.concatenate([v[:, 0:HH], v[:, 2 * HH:3 * HH]], axis=1)
    hi = jnp.concatenate([v[:, HH:2 * HH], v[:, 3 * HH:4 * HH]], axis=1)
    wl = lax.bitcast_convert_type(lo, jnp.int32)
    wh = lax.bitcast_convert_type(hi, jnp.int32)
    rl = wl + jnp.int32(0x7FFF) + ((wl >> 16) & 1)
    rh = wh + jnp.int32(0x7FFF) + ((wh >> 16) & 1)
    word = ((rl >> 16) & jnp.int32(0xFFFF)) | (rh & _mask16())
    return lax.bitcast_convert_type(word, jnp.float32)


def _tables_body(x_ref, wa_ref, wb_ref, ba_ref, a_ref, b_ref):
    xb = x_ref[...]
    a = jnp.dot(xb, wa_ref[...], preferred_element_type=jnp.float32) + ba_ref[...]
    bt = jnp.dot(xb, wb_ref[...], preferred_element_type=jnp.float32)
    a_ref[...] = _pack_bf16_pairs(a)
    b_ref[...] = _pack_bf16_pairs(bt)


def _tables(x2, WA, WB, bA):
    BN = 2000
    return pl.pallas_call(
        _tables_body,
        grid=(N // BN,),
        in_specs=[
            pl.BlockSpec((BN, XD), lambda i: (i, 0)),
            pl.BlockSpec((XD, GW), lambda i: (0, 0)),
            pl.BlockSpec((XD, GW), lambda i: (0, 0)),
            pl.BlockSpec((1, GW), lambda i: (0, 0)),
        ],
        out_specs=[
            pl.BlockSpec((BN, GWP), lambda i: (i, 0)),
            pl.BlockSpec((BN, GWP), lambda i: (i, 0)),
        ],
        out_shape=[jax.ShapeDtypeStruct((N, GWP), jnp.float32)] * 2,
    )(x2, WA, WB, bA)


def _sc_gather(A, B, ei_flat):
    mesh = plsc.VectorSubcoreMesh(core_axis_name="c", subcore_axis_name="s")

    @functools.partial(
        pl.kernel,
        mesh=mesh,
        out_type=(jax.ShapeDtypeStruct((E, GWP), jnp.float32),
                  jax.ShapeDtypeStruct((E, GWP), jnp.float32)),
        scratch_types=[
            pltpu.VMEM((2, KG), jnp.int32),
            pltpu.VMEM((2, KG), jnp.int32),
            pltpu.VMEM((2, KG, GWP), jnp.float32),
            pltpu.VMEM((2, KG, GWP), jnp.float32),
            [pltpu.SemaphoreType.DMA] * 2,
            [pltpu.SemaphoreType.DMA] * 2,
            [pltpu.SemaphoreType.DMA] * 2,
            [pltpu.SemaphoreType.DMA] * 2,
        ],
    )
    def k(a_hbm, b_hbm, ei_hbm, outa_hbm, outb_hbm,
          idxr, idxs, bufa, bufb, semi, sema, semb, semo):
        wid = lax.axis_index("s") * NC + lax.axis_index("c")
        base = wid * EPW

        def issue_idx(c, b):
            pltpu.async_copy(ei_hbm.at[pl.ds(E + base + c * KG, KG)],
                             idxr.at[b], semi[b])
            pltpu.async_copy(ei_hbm.at[pl.ds(base + c * KG, KG)],
                             idxs.at[b], semi[b])

        issue_idx(0, 0)
        issue_idx(1, 1)

        def pair(p, carry):
            for b in range(2):
                c = p * 2 + b
                pltpu.make_async_copy(ei_hbm.at[pl.ds(0, KG)],
                                      idxr.at[b], semi[b]).wait()
                pltpu.make_async_copy(ei_hbm.at[pl.ds(0, KG)],
                                      idxs.at[b], semi[b]).wait()

                @pl.when(c >= 2)
                def _():
                    pltpu.make_async_copy(
                        bufa.at[b], outa_hbm.at[pl.ds(base, KG)],
                        semo[b]).wait()
                    pltpu.make_async_copy(
                        bufb.at[b], outb_hbm.at[pl.ds(base, KG)],
                        semo[b]).wait()

                ca = pltpu.async_copy(a_hbm.at[idxr.at[b]], bufa.at[b],
                                      sema[b])
                cb = pltpu.async_copy(b_hbm.at[idxs.at[b]], bufb.at[b],
                                      semb[b])

                @pl.when(c + 2 < GCH)
                def _():
                    issue_idx(c + 2, b)

                ca.wait()
                cb.wait()
                pltpu.async_copy(bufa.at[b],
                                 outa_hbm.at[pl.ds(base + c * KG, KG)],
                                 semo[b])
                pltpu.async_copy(bufb.at[b],
                                 outb_hbm.at[pl.ds(base + c * KG, KG)],
                                 semo[b])
            return carry

        lax.fori_loop(0, GCH // 2, pair, 0)
        for b in range(2):
            pltpu.make_async_copy(bufa.at[b], outa_hbm.at[pl.ds(base, KG)],
                                  semo[b]).wait()
            pltpu.make_async_copy(bufb.at[b], outb_hbm.at[pl.ds(base, KG)],
                                  semo[b]).wait()

        @pl.when(wid <= 1)
        def _tail_full():
            toff = ETAIL + wid * KG
            pltpu.sync_copy(ei_hbm.at[pl.ds(E + toff, KG)], idxr.at[0])
            pltpu.sync_copy(ei_hbm.at[pl.ds(toff, KG)], idxs.at[0])
            pltpu.async_copy(a_hbm.at[idxr.at[0]], bufa.at[0], sema[0]).wait()
            pltpu.async_copy(b_hbm.at[idxs.at[0]], bufb.at[0], semb[0]).wait()
            pltpu.sync_copy(bufa.at[0], outa_hbm.at[pl.ds(toff, KG)])
            pltpu.sync_copy(bufb.at[0], outb_hbm.at[pl.ds(toff, KG)])

        @pl.when(wid == 2)
        def _tail_rem():
            KT = 48
            toff = ETAIL + 2 * KG
            pltpu.sync_copy(ei_hbm.at[pl.ds(E + toff, KT)],
                            idxr.at[0, pl.ds(0, KT)])
            pltpu.sync_copy(ei_hbm.at[pl.ds(toff, KT)],
                            idxs.at[0, pl.ds(0, KT)])
            pltpu.async_copy(a_hbm.at[idxr.at[0, pl.ds(0, KT)]],
                             bufa.at[0, pl.ds(0, KT)], sema[0]).wait()
            pltpu.async_copy(b_hbm.at[idxs.at[0, pl.ds(0, KT)]],
                             bufb.at[0, pl.ds(0, KT)], semb[0]).wait()
            pltpu.sync_copy(bufa.at[0, pl.ds(0, KT)],
                            outa_hbm.at[pl.ds(toff, KT)])
            pltpu.sync_copy(bufb.at[0, pl.ds(0, KT)],
                            outb_hbm.at[pl.ds(toff, KT)])

    return k(A, B, ei_flat)


def _unpack_pair(ref):
    w = lax.bitcast_convert_type(ref[...], jnp.int32)
    lo = lax.bitcast_convert_type(w << 16, jnp.float32)
    hi = lax.bitcast_convert_type(w & _mask16(), jnp.float32)
    return lo, hi


def _edge_body(ga_ref, gb_ref, e_ref, we_ref, pw2_ref, pb2_ref, f_ref,
               fw2_ref, fb2_ref, eout_ref, mt_ref):
    lo_a, hi_a = _unpack_pair(ga_ref)
    lo_b, hi_b = _unpack_pair(gb_ref)
    lo = lo_a + lo_b
    hi = hi_a + hi_b
    HH = H // 2
    g1 = jnp.concatenate([lo[:, :HH], hi[:, :HH]], axis=1)
    g2 = jnp.concatenate([lo[:, HH:], hi[:, HH:]], axis=1)
    h1 = jnp.maximum(
        g1 + jnp.dot(e_ref[...], we_ref[...], preferred_element_type=jnp.float32),
        0.0)
    eo = jnp.dot(h1, pw2_ref[...], preferred_element_type=jnp.float32) + pb2_ref[...]
    eout_ref[...] = eo
    h2 = jnp.maximum(
        g2 + jnp.dot(eo, f_ref[...], preferred_element_type=jnp.float32),
        0.0)
    mt = lax.dot_general(fw2_ref[...], h2, (((0,), (1,)), ((), ())),
                         preferred_element_type=jnp.float32)
    mt_ref[...] = mt + fb2_ref[...]


def _edge_mlp(Ga, Gb, e2, We, pW2, pb2, F, fW2, fb2col):
    TE = 1280
    return pl.pallas_call(
        _edge_body,
        grid=(E // TE,),
        in_specs=[
            pl.BlockSpec((TE, GWP), lambda i: (i, 0)),
            pl.BlockSpec((TE, GWP), lambda i: (i, 0)),
            pl.BlockSpec((TE, ED), lambda i: (i, 0)),
            pl.BlockSpec((ED, H), lambda i: (0, 0)),
            pl.BlockSpec((H, ED), lambda i: (0, 0)),
            pl.BlockSpec((1, ED), lambda i: (0, 0)),
            pl.BlockSpec((ED, H), lambda i: (0, 0)),
            pl.BlockSpec((H, H), lambda i: (0, 0)),
            pl.BlockSpec((H, 1), lambda i: (0, 0)),
        ],
        out_specs=[
            pl.BlockSpec((TE, ED), lambda i: (i, 0)),
            pl.BlockSpec((H, TE), lambda i: (0, i)),
        ],
        out_shape=[
            jax.ShapeDtypeStruct((E, ED), jnp.float32),
            jax.ShapeDtypeStruct((H, E), jnp.float32),
        ],
    )(Ga, Gb, e2, We, pW2, pb2, F, fW2, fb2col)


def _sc_scatter(mT, ei_flat):
    mesh = plsc.VectorSubcoreMesh(core_axis_name="c", subcore_axis_name="s")

    @functools.partial(
        pl.kernel,
        mesh=mesh,
        out_type=jax.ShapeDtypeStruct((H * NP,), jnp.float32),
        compiler_params=pltpu.CompilerParams(needs_layout_passes=False),
        scratch_types=[
            pltpu.VMEM((2, KS), jnp.int32),
            pltpu.VMEM((2, CPW, KS), jnp.float32),
            pltpu.VMEM((CPW * NP,), jnp.float32),
            [pltpu.SemaphoreType.DMA] * 2,
            [pltpu.SemaphoreType.DMA] * 2,
        ],
    )
    def k(mt_hbm, r_hbm, aggt_hbm, idxb, mbuf, acc, semi, semm):
        wid = lax.axis_index("s") * NC + lax.axis_index("c")
        zeros = jnp.zeros((16,), jnp.float32)

        def issue(c, b):
            pltpu.async_copy(r_hbm.at[pl.ds(E + c * KS, KS)], idxb.at[b],
                             semi[b])
            pltpu.async_copy(
                mt_hbm.at[pl.ds(wid * CPW, CPW), pl.ds(c * KS, KS)],
                mbuf.at[b], semm[b])

        issue(0, 0)
        issue(1, 1)

        def zchunk(j, c2):
            for v in range(16):
                acc[pl.ds((j * 16 + v) * 16, 16)] = zeros
            return c2

        lax.fori_loop(0, CPW * NP // 256, zchunk, 0)

        def pair(p, carry):
            for b in range(2):
                c = p * 2 + b
                pltpu.make_async_copy(r_hbm.at[pl.ds(0, KS)], idxb.at[b],
                                      semi[b]).wait()
                pltpu.make_async_copy(
                    mt_hbm.at[pl.ds(0, CPW), pl.ds(0, KS)], mbuf.at[b],
                    semm[b]).wait()

                @plsc.parallel_loop(0, KS // 16, unroll=4)
                def group(g):
                    ids = idxb[b, pl.ds(g * 16, 16)]
                    for ci in range(CPW):
                        data = mbuf[b, ci, pl.ds(g * 16, 16)]
                        plsc.addupdate_scatter(acc, [ids + (ci * NP)], data)

                @pl.when(c + 2 < SCH)
                def _():
                    issue(c + 2, b)
            return carry

        lax.fori_loop(0, SCH // 2, pair, 0)
        pltpu.sync_copy(acc, aggt_hbm.at[pl.ds(wid * CPW * NP, CPW * NP)])

    return k(mT, ei_flat)


def _node_body(x_ref, aggt_ref, na_ref, nb_ref, b1_ref, w2_ref, b2_ref,
               out_ref):
    agg_contrib = lax.dot_general(aggt_ref[...], nb_ref[...],
                                  (((0,), (0,)), ((), ())),
                                  preferred_element_type=jnp.float32)
    h = jnp.maximum(
        jnp.dot(x_ref[...], na_ref[...], preferred_element_type=jnp.float32)
        + agg_contrib + b1_ref[...], 0.0)
    out_ref[...] = jnp.dot(h, w2_ref[...], preferred_element_type=jnp.float32) + b2_ref[...]


def _node_mlp(x2, aggT, nuA, nuB, nb1, nW2, nb2):
    BN = 2048
    return pl.pallas_call(
        _node_body,
        grid=(NP // BN,),
        in_specs=[
            pl.BlockSpec((BN, XD), lambda i: (i, 0)),
            pl.BlockSpec((H, BN), lambda i: (0, i)),
            pl.BlockSpec((XD, H), lambda i: (0, 0)),
            pl.BlockSpec((H, H), lambda i: (0, 0)),
            pl.BlockSpec((1, H), lambda i: (0, 0)),
            pl.BlockSpec((H, XD), lambda i: (0, 0)),
            pl.BlockSpec((1, XD), lambda i: (0, 0)),
        ],
        out_specs=pl.BlockSpec((BN, XD), lambda i: (i, 0)),
        out_shape=jax.ShapeDtypeStruct((N, XD), jnp.float32),
    )(x2, aggT, nuA, nuB, nb1, nW2, nb2)


def kernel(x, e, edge_index, psi_W1, psi_b1, psi_W2, psi_b2,
           phi_W1, phi_b1, phi_W2, phi_b2, nu_W1, nu_b1, nu_W2, nu_b2):
    x2 = x.reshape(N, XD)
    e2 = e.reshape(E, ED)
    ei_flat = edge_index.astype(jnp.int32).reshape(2 * E)

    We = psi_W1[:ED]
    WA = jnp.concatenate([psi_W1[ED:ED + XD], phi_W1[:XD]], axis=1)
    WB = jnp.concatenate([psi_W1[ED + XD:], phi_W1[XD:2 * XD]], axis=1)
    bA = jnp.concatenate([psi_b1, phi_b1]).reshape(1, GW)
    F = phi_W1[2 * XD:]

    A, B = _tables(x2, WA, WB, bA)
    Ga, Gb = _sc_gather(A, B, ei_flat)
    e_out, mT = _edge_mlp(Ga, Gb, e2, We, psi_W2, psi_b2.reshape(1, ED),
                          F, phi_W2, phi_b2.reshape(H, 1))
    aggT = _sc_scatter(mT, ei_flat).reshape(H, NP)
    x_out = _node_mlp(x2, aggT, nu_W1[:XD], nu_W1[XD:], nu_b1.reshape(1, H),
                      nu_W2, nu_b2.reshape(1, XD))
    return (x_out.reshape(1, N, XD), e_out.reshape(1, E, ED))
